# Initial kernel scaffold; baseline (speedup 1.0000x reference)
import jax
import jax.numpy as jnp
from jax.experimental import pallas as pl


def kernel(user, item, q, indicator, user_table, item_table, skill_table, W_stu, a_stu, W_item_stu, W_item_skill, a_item_stu, a_item_skill, a_map_stu, a_map_skill, W_skill_item, a_skill_item, W_fuse_stu, b_fuse_stu, W_fuse_item, b_fuse_item, W_pred, b_pred):
    raise NotImplementedError("write your pallas kernel here")


# speedup vs baseline: 1.3238x; 1.3238x over previous
import functools

import jax
import jax.numpy as jnp
from jax import lax
from jax.experimental import pallas as pl
from jax.experimental.pallas import tpu as pltpu
from jax.experimental.pallas import tpu_sc as plsc

_EPS = 1e-9
_F32 = jnp.float32


def _nt(x, w):
    return lax.dot_general(x, w, (((1,), (1,)), ((), ())),
                           preferred_element_type=_F32)


def _nn(x, w):
    return lax.dot_general(x, w, (((1,), (0,)), ((), ())),
                           preferred_element_type=_F32)


def _k1_body(item_ref, user_ref, wstu_ref, astu2_ref, wis_ref, ais2_ref,
             wu_ref, eui_ref, wv_ref, eis_ref):
    ti = _nt(item_ref[...], wstu_ref[...])
    beta = _nn(ti, astu2_ref[...])
    e_ui = jnp.exp(beta - jnp.max(beta))
    wu_ref[...] = e_ui * ti
    eui_ref[...] = e_ui
    tsu = _nt(user_ref[...], wis_ref[...])
    gam = _nn(tsu, ais2_ref[...])
    e_is = jnp.exp(gam - jnp.max(gam))
    wv_ref[...] = e_is * tsu
    eis_ref[...] = e_is


def _k2_body(ind_ref, ut_ref, wu_ref, eui_ref, wv_ref, eis_ref, wf1_ref,
             hu_ref, nit_ref, si_ref):
    step = pl.program_id(0)
    ind = ind_ref[...]
    nu = _nn(ind, wu_ref[...])
    su = jnp.sum(ind * eui_ref[...], axis=1, keepdims=True)
    fu = ut_ref[...] + nu / (su + _EPS)
    hu_ref[...] = _nt(fu, wf1_ref[...])
    nit_c = lax.dot_general(wv_ref[...], ind, (((0,), (0,)), ((), ())),
                            preferred_element_type=_F32)
    si_c = jnp.sum(ind * eis_ref[...], axis=0, keepdims=True)

    @pl.when(step == 0)
    def _init():
        nit_ref[...] = nit_c
        si_ref[...] = si_c

    @pl.when(step != 0)
    def _acc():
        nit_ref[...] += nit_c
        si_ref[...] += si_c


def _k3_body(nit_ref, si_ref, item_ref, skill_ref, q_ref,
             wik_ref, aik2_ref, wsi_ref, asi2_ref,
             ams1_ref, ams2_ref, amk1_ref, amk2_ref,
             wfs2_ref, bfs_ref, wfi1_ref, wfi2_ref, bfi_ref, wp_ref,
             t_ref):
    it = item_ref[...]
    q = q_ref[...]
    stu_t = nit_ref[...] / (si_ref[...] + _EPS)
    stu = lax.transpose(stu_t, (1, 0))
    tsk = _nt(skill_ref[...], wik_ref[...])
    dlt = lax.dot_general(aik2_ref[...], tsk, (((1,), (1,)), ((), ())),
                          preferred_element_type=_F32)
    e_ik = jnp.exp(dlt - jnp.max(dlt))
    qe = q * e_ik
    sk_num = _nn(qe, tsk)
    sk_den = jnp.sum(qe, axis=1, keepdims=True)
    skill_fused = sk_num / (sk_den + _EPS)
    ms = _nn(it, ams1_ref[...]) + _nn(stu, ams2_ref[...])
    mk = _nn(it, amk1_ref[...]) + _nn(skill_fused, amk2_ref[...])
    w0 = jax.nn.sigmoid(ms - mk)
    final_item = it + w0 * stu + (1.0 - w0) * skill_fused
    tis = _nt(it, wsi_ref[...])
    epsv = _nn(tis, asi2_ref[...])
    e_si = jnp.exp(epsv - jnp.max(epsv))
    ns = lax.dot_general(q, e_si * tis, (((0,), (0,)), ((), ())),
                         preferred_element_type=_F32)
    ss = lax.dot_general(q, e_si, (((0,), (0,)), ((), ())),
                         preferred_element_type=_F32)
    final_skill = skill_ref[...] + ns / (ss + _EPS)
    qs = jnp.sum(q, axis=1, keepdims=True)
    se = _nn(q, final_skill) / (qs + _EPS)
    gs = _nt(se, wfs2_ref[...]) + bfs_ref[...]
    hi = (_nt(final_item, wfi1_ref[...]) + _nt(se, wfi2_ref[...])
          + bfi_ref[...])
    zi = _nn(jax.nn.sigmoid(hi), wp_ref[...])
    pad = jnp.zeros((q.shape[0], 127), _F32)
    t_ref[...] = jnp.concatenate([gs, zi, pad], axis=1)


def _k5_body(hug_ref, tg_ref, wp_ref, bp_ref, out_ref):
    d = 128
    t = tg_ref[...]
    hs = jax.nn.sigmoid(hug_ref[...] + t[:, :d])
    acc = _nn(hs, wp_ref[...])
    out_ref[...] = jax.nn.sigmoid(acc - t[:, d:d + 1] + bp_ref[...])


def _make_gather(U, I, B, D1, D2):
    info = plsc.get_sparse_core_info()
    nc, ns = info.num_cores, info.num_subcores
    nw = nc * ns
    bpw = B // nw
    mesh = plsc.VectorSubcoreMesh(core_axis_name="c", subcore_axis_name="s")

    @functools.partial(
        pl.kernel,
        out_type=[jax.ShapeDtypeStruct((B, D1), _F32),
                  jax.ShapeDtypeStruct((B, D2), _F32)],
        mesh=mesh,
        scratch_types=[
            pltpu.VMEM((bpw,), jnp.int32),
            pltpu.VMEM((bpw,), jnp.int32),
            pltpu.VMEM((bpw, D1), _F32),
            pltpu.VMEM((bpw, D2), _F32),
            pltpu.SemaphoreType.DMA,
        ],
    )
    def gather(uidx_hbm, iidx_hbm, hu_hbm, ti_hbm, outu_hbm, outi_hbm,
               uidx_v, iidx_v, hurows_v, tirows_v, sem):
        wid = lax.axis_index("s") * nc + lax.axis_index("c")
        base = wid * bpw
        pltpu.sync_copy(uidx_hbm.at[pl.ds(base, bpw)], uidx_v)
        pltpu.sync_copy(iidx_hbm.at[pl.ds(base, bpw)], iidx_v)
        pltpu.async_copy(hu_hbm.at[uidx_v], hurows_v, sem).wait()
        pltpu.async_copy(ti_hbm.at[iidx_v], tirows_v, sem).wait()
        pltpu.sync_copy(hurows_v, outu_hbm.at[pl.ds(base, bpw)])
        pltpu.sync_copy(tirows_v, outi_hbm.at[pl.ds(base, bpw)])

    return gather


def kernel(user, item, q, indicator, user_table, item_table, skill_table,
           W_stu, a_stu, W_item_stu, W_item_skill, a_item_stu, a_item_skill,
           a_map_stu, a_map_skill, W_skill_item, a_skill_item,
           W_fuse_stu, b_fuse_stu, W_fuse_item, b_fuse_item, W_pred, b_pred):
    U, d = user_table.shape
    I = item_table.shape[0]
    S = skill_table.shape[0]
    B = user.shape[0]
    BU = 1000
    col = lambda v: v.reshape(-1, 1).astype(_F32)
    row = lambda v: v.reshape(1, -1).astype(_F32)

    wu, e_ui, wv, e_is = pl.pallas_call(
        _k1_body,
        out_shape=[jax.ShapeDtypeStruct((I, d), _F32),
                   jax.ShapeDtypeStruct((I, 1), _F32),
                   jax.ShapeDtypeStruct((U, d), _F32),
                   jax.ShapeDtypeStruct((U, 1), _F32)],
    )(item_table, user_table, W_stu, col(a_stu[d:]),
      W_item_stu, col(a_item_stu[d:]))

    nsteps = U // BU
    full = lambda shape: pl.BlockSpec(shape, lambda i: (0, 0))
    blk = lambda shape: pl.BlockSpec(shape, lambda i: (i, 0))
    hu, nit, si = pl.pallas_call(
        _k2_body,
        grid=(nsteps,),
        in_specs=[blk((BU, I)), blk((BU, d)), full((I, d)), full((1, I)),
                  blk((BU, d)), blk((BU, 1)), full((d, d))],
        out_specs=[blk((BU, d)), full((d, I)), full((1, I))],
        out_shape=[jax.ShapeDtypeStruct((U, d), _F32),
                   jax.ShapeDtypeStruct((d, I), _F32),
                   jax.ShapeDtypeStruct((1, I), _F32)],
    )(indicator, user_table, wu, e_ui.reshape(1, I), wv, e_is,
      W_fuse_stu[:, :d])

    t_item = pl.pallas_call(
        _k3_body,
        out_shape=jax.ShapeDtypeStruct((I, 2 * d), _F32),
    )(nit, si, item_table, skill_table, q,
      W_item_skill, row(a_item_skill[d:]), W_skill_item, col(a_skill_item[d:]),
      col(a_map_stu[:d]), col(a_map_stu[d:]),
      col(a_map_skill[:d]), col(a_map_skill[d:]),
      W_fuse_stu[:, d:], row(b_fuse_stu),
      W_fuse_item[:, :d], W_fuse_item[:, d:], row(b_fuse_item),
      col(W_pred[0]))

    hug, tg = _make_gather(U, I, B, d, 2 * d)(
        user.astype(jnp.int32), item.astype(jnp.int32), hu, t_item)

    pred = pl.pallas_call(
        _k5_body,
        out_shape=jax.ShapeDtypeStruct((B, 1), _F32),
    )(hug, tg, col(W_pred[0]), b_pred.reshape(1, 1))
    return pred.reshape(B)


# speedup vs baseline: 1.4736x; 1.1131x over previous
import functools

import jax
import jax.numpy as jnp
from jax import lax
from jax.experimental import pallas as pl
from jax.experimental.pallas import tpu as pltpu
from jax.experimental.pallas import tpu_sc as plsc

_EPS = 1e-9
_F32 = jnp.float32


def _nt(x, w):
    return lax.dot_general(x, w, (((1,), (1,)), ((), ())),
                           preferred_element_type=_F32)


def _nn(x, w):
    return lax.dot_general(x, w, (((1,), (0,)), ((), ())),
                           preferred_element_type=_F32)


def _tn(x, w):
    return lax.dot_general(x, w, (((0,), (0,)), ((), ())),
                           preferred_element_type=_F32)


_A_STU2, _A_IS2, _A_IK2, _A_SI2 = 0, 1, 2, 3
_AMS1, _AMS2, _AMK1, _AMK2 = 4, 5, 6, 7
_B_FS, _B_FI, _W_PRED, _B_PRED = 8, 9, 10, 11


def _k123_body(ind_ref, ut_ref, item_ref, skill_ref, q_ref,
               wstu_ref, wis_ref, wik_ref, wsi_ref, wfs_ref, wfi_ref,
               vecs_ref, hu_ref, t_ref, wu_s, eui_s, nit_s, si_s):
    step = pl.program_id(0)
    d = 128
    vecs = vecs_ref[...]

    @pl.when(step == 0)
    def _prep():
        ti = _nt(item_ref[...], wstu_ref[...])
        beta = _nt(ti, vecs[_A_STU2:_A_STU2 + 1, :])
        wu_s[...] = jnp.exp(beta) * ti
        eui_s[...] = jnp.exp(_nt(vecs[_A_STU2:_A_STU2 + 1, :], ti))

    ind = ind_ref[...]
    ut = ut_ref[...]
    nu = _nn(ind, wu_s[...])
    su = jnp.sum(ind * eui_s[...], axis=1, keepdims=True)
    fu = ut + nu / (su + _EPS)
    hu_ref[...] = _nt(fu, wfs_ref[:, :d])
    tsu = _nt(ut, wis_ref[...])
    e_is = jnp.exp(_nt(tsu, vecs[_A_IS2:_A_IS2 + 1, :]))
    nit_c = _tn(e_is * tsu, ind)
    si_c = jnp.sum(ind * e_is, axis=0, keepdims=True)

    @pl.when(step == 0)
    def _init():
        nit_s[...] = nit_c
        si_s[...] = si_c

    @pl.when(step != 0)
    def _acc():
        nit_s[...] += nit_c
        si_s[...] += si_c

    @pl.when(step == pl.num_programs(0) - 1)
    def _finalize():
        it = item_ref[...]
        q = q_ref[...]
        stu = lax.transpose(nit_s[...] / (si_s[...] + _EPS), (1, 0))
        tsk = _nt(skill_ref[...], wik_ref[...])
        e_ik = jnp.exp(_nt(vecs[_A_IK2:_A_IK2 + 1, :], tsk))
        qe = q * e_ik
        skill_fused = _nn(qe, tsk) / (jnp.sum(qe, axis=1, keepdims=True)
                                      + _EPS)
        ms = (_nt(it, vecs[_AMS1:_AMS1 + 1, :])
              + _nt(stu, vecs[_AMS2:_AMS2 + 1, :]))
        mk = (_nt(it, vecs[_AMK1:_AMK1 + 1, :])
              + _nt(skill_fused, vecs[_AMK2:_AMK2 + 1, :]))
        w0 = jax.nn.sigmoid(ms - mk)
        final_item = it + w0 * stu + (1.0 - w0) * skill_fused
        tis = _nt(it, wsi_ref[...])
        e_si = jnp.exp(_nt(tis, vecs[_A_SI2:_A_SI2 + 1, :]))
        final_skill = (skill_ref[...]
                       + _tn(q, e_si * tis) / (_tn(q, e_si) + _EPS))
        qs = jnp.sum(q, axis=1, keepdims=True)
        se = _nn(q, final_skill) / (qs + _EPS)
        gs = _nt(se, wfs_ref[:, d:]) + vecs[_B_FS:_B_FS + 1, :]
        hi = (_nt(final_item, wfi_ref[:, :d]) + _nt(se, wfi_ref[:, d:])
              + vecs[_B_FI:_B_FI + 1, :])
        zi = _nt(jax.nn.sigmoid(hi), vecs[_W_PRED:_W_PRED + 1, :])
        pad = jnp.zeros((q.shape[0], d - 1), _F32)
        t_ref[...] = jnp.concatenate([gs, zi, pad], axis=1)


def _k5_body(hug_ref, tg_ref, vecs_ref, out_ref):
    d = 128
    t = tg_ref[...]
    hs = jax.nn.sigmoid(hug_ref[...] + t[:, :d])
    acc = _nt(hs, vecs_ref[_W_PRED:_W_PRED + 1, :])
    bp = vecs_ref[_B_PRED:_B_PRED + 1, 0:1]
    out_ref[...] = jax.nn.sigmoid(acc - t[:, d:d + 1] + bp)


def _make_gather(B, D1, D2):
    info = plsc.get_sparse_core_info()
    nc, ns = info.num_cores, info.num_subcores
    bpw = B // (nc * ns)
    mesh = plsc.VectorSubcoreMesh(core_axis_name="c", subcore_axis_name="s")

    @functools.partial(
        pl.kernel,
        out_type=[jax.ShapeDtypeStruct((B, D1), _F32),
                  jax.ShapeDtypeStruct((B, D2), _F32)],
        mesh=mesh,
        scratch_types=[
            pltpu.VMEM((bpw,), jnp.int32),
            pltpu.VMEM((bpw,), jnp.int32),
            pltpu.VMEM((bpw, D1), _F32),
            pltpu.VMEM((bpw, D2), _F32),
            pltpu.SemaphoreType.DMA,
        ],
    )
    def gather(uidx_hbm, iidx_hbm, hu_hbm, ti_hbm, outu_hbm, outi_hbm,
               uidx_v, iidx_v, hurows_v, tirows_v, sem):
        wid = lax.axis_index("s") * nc + lax.axis_index("c")
        base = wid * bpw
        pltpu.sync_copy(uidx_hbm.at[pl.ds(base, bpw)], uidx_v)
        pltpu.sync_copy(iidx_hbm.at[pl.ds(base, bpw)], iidx_v)
        pltpu.async_copy(hu_hbm.at[uidx_v], hurows_v, sem).wait()
        pltpu.async_copy(ti_hbm.at[iidx_v], tirows_v, sem).wait()
        pltpu.sync_copy(hurows_v, outu_hbm.at[pl.ds(base, bpw)])
        pltpu.sync_copy(tirows_v, outi_hbm.at[pl.ds(base, bpw)])

    return gather


def kernel(user, item, q, indicator, user_table, item_table, skill_table,
           W_stu, a_stu, W_item_stu, W_item_skill, a_item_stu, a_item_skill,
           a_map_stu, a_map_skill, W_skill_item, a_skill_item,
           W_fuse_stu, b_fuse_stu, W_fuse_item, b_fuse_item, W_pred, b_pred):
    U, d = user_table.shape
    I = item_table.shape[0]
    B = user.shape[0]
    BU = 1000

    vecs = jnp.stack([
        a_stu[d:], a_item_stu[d:], a_item_skill[d:], a_skill_item[d:],
        a_map_stu[:d], a_map_stu[d:], a_map_skill[:d], a_map_skill[d:],
        b_fuse_stu, b_fuse_item, W_pred[0],
        jnp.broadcast_to(b_pred, (d,)),
    ]).astype(_F32)

    nsteps = U // BU
    full = lambda shape: pl.BlockSpec(shape, lambda i: (0, 0))
    blk = lambda shape: pl.BlockSpec(shape, lambda i: (i, 0))
    hu, t_item = pl.pallas_call(
        _k123_body,
        grid=(nsteps,),
        in_specs=[blk((BU, I)), blk((BU, d)), full((I, d)), full((d, d)),
                  full((I, d)), full((d, d)), full((d, d)), full((d, d)),
                  full((d, d)), full((d, 2 * d)), full((d, 2 * d)),
                  full((12, d))],
        out_specs=[blk((BU, d)), full((I, 2 * d))],
        out_shape=[jax.ShapeDtypeStruct((U, d), _F32),
                   jax.ShapeDtypeStruct((I, 2 * d), _F32)],
        scratch_shapes=[pltpu.VMEM((I, d), _F32), pltpu.VMEM((1, I), _F32),
                        pltpu.VMEM((d, I), _F32), pltpu.VMEM((1, I), _F32)],
    )(indicator, user_table, item_table, skill_table, q,
      W_stu, W_item_stu, W_item_skill, W_skill_item, W_fuse_stu, W_fuse_item,
      vecs)

    hug, tg = _make_gather(B, d, 2 * d)(
        user.astype(jnp.int32), item.astype(jnp.int32), hu, t_item)

    pred = pl.pallas_call(
        _k5_body,
        out_shape=jax.ShapeDtypeStruct((B, 1), _F32),
    )(hug, tg, vecs)
    return pred.reshape(B)


# speedup vs baseline: 1.5202x; 1.0316x over previous
import functools

import jax
import jax.numpy as jnp
from jax import lax
from jax.experimental import pallas as pl
from jax.experimental.pallas import tpu as pltpu
from jax.experimental.pallas import tpu_sc as plsc

_EPS = 1e-9
_F32 = jnp.float32


def _nt(x, w):
    return lax.dot_general(x, w, (((1,), (1,)), ((), ())),
                           preferred_element_type=_F32)


def _nn(x, w):
    return lax.dot_general(x, w, (((1,), (0,)), ((), ())),
                           preferred_element_type=_F32)


def _tn(x, w):
    return lax.dot_general(x, w, (((0,), (0,)), ((), ())),
                           preferred_element_type=_F32)


_A_STU2, _A_IS2, _A_IK2, _A_SI2 = 0, 1, 2, 3
_AMS1, _AMS2, _AMK1, _AMK2 = 4, 5, 6, 7
_B_FS, _B_FI, _W_PRED, _B_PRED = 8, 9, 10, 11


def _k123_body(inda_ref, indb_ref, ut_ref, item_ref, skill_ref, q_ref,
               wstu_ref, wis_ref, wik_ref, wsi_ref, wfs_ref, wfi_ref,
               vecs_ref, hu_ref, t_ref, wu_s, nit_s):
    step = pl.program_id(0)
    d = 128
    h = inda_ref.shape[0]
    vecs = vecs_ref[...]

    @pl.when(step == 0)
    def _prep():
        ti = _nt(item_ref[...], wstu_ref[...])
        beta = _nt(ti, vecs[_A_STU2:_A_STU2 + 1, :])
        e_ui = jnp.exp(beta)
        wu_s[...] = jnp.concatenate(
            [e_ui * ti, e_ui, jnp.zeros((ti.shape[0], d - 1), _F32)],
            axis=1).astype(jnp.bfloat16)

    ut = ut_ref[...]
    tsu = _nt(ut, wis_ref[...])
    e_is = jnp.exp(_nt(tsu, vecs[_A_IS2:_A_IS2 + 1, :]))
    lhs = jnp.concatenate(
        [e_is * tsu, e_is, jnp.zeros((tsu.shape[0], d - 1), _F32)],
        axis=1).astype(jnp.bfloat16)

    nit_parts = []
    for k, ref in ((0, inda_ref), (1, indb_ref)):
        ind_b = ref[...].astype(jnp.bfloat16)
        nu_su = _nn(ind_b, wu_s[...])
        fu = (ut[k * h:(k + 1) * h, :]
              + nu_su[:, :d] / (nu_su[:, d:d + 1] + _EPS))
        hu_ref[k * h:(k + 1) * h, :] = _nt(fu, wfs_ref[:, :d])
        nit_parts.append(_tn(lhs[k * h:(k + 1) * h, :], ind_b))
    nit_c = nit_parts[0] + nit_parts[1]

    @pl.when(step == 0)
    def _init():
        nit_s[...] = nit_c

    @pl.when(step != 0)
    def _acc():
        nit_s[...] += nit_c

    @pl.when(step == pl.num_programs(0) - 1)
    def _finalize():
        it = item_ref[...]
        q = q_ref[...]
        stu = lax.transpose(
            nit_s[:d, :] / (nit_s[d:d + 1, :] + _EPS), (1, 0))
        tsk = _nt(skill_ref[...], wik_ref[...])
        e_ik = jnp.exp(_nt(vecs[_A_IK2:_A_IK2 + 1, :], tsk))
        qe = q * e_ik
        skill_fused = _nn(qe, tsk) / (jnp.sum(qe, axis=1, keepdims=True)
                                      + _EPS)
        ms = (_nt(it, vecs[_AMS1:_AMS1 + 1, :])
              + _nt(stu, vecs[_AMS2:_AMS2 + 1, :]))
        mk = (_nt(it, vecs[_AMK1:_AMK1 + 1, :])
              + _nt(skill_fused, vecs[_AMK2:_AMK2 + 1, :]))
        w0 = jax.nn.sigmoid(ms - mk)
        final_item = it + w0 * stu + (1.0 - w0) * skill_fused
        tis = _nt(it, wsi_ref[...])
        e_si = jnp.exp(_nt(tis, vecs[_A_SI2:_A_SI2 + 1, :]))
        final_skill = (skill_ref[...]
                       + _tn(q, e_si * tis) / (_tn(q, e_si) + _EPS))
        qs = jnp.sum(q, axis=1, keepdims=True)
        se = _nn(q, final_skill) / (qs + _EPS)
        gs = _nt(se, wfs_ref[:, d:]) + vecs[_B_FS:_B_FS + 1, :]
        hi = (_nt(final_item, wfi_ref[:, :d]) + _nt(se, wfi_ref[:, d:])
              + vecs[_B_FI:_B_FI + 1, :])
        zi = _nt(jax.nn.sigmoid(hi), vecs[_W_PRED:_W_PRED + 1, :])
        pad = jnp.zeros((q.shape[0], d - 1), _F32)
        t_ref[...] = jnp.concatenate([gs, zi, pad], axis=1)


def _make_sc_gather(B, D1, D2):
    info = plsc.get_sparse_core_info()
    nc, ns = info.num_cores, info.num_subcores
    bpw = B // (nc * ns)
    mesh = plsc.VectorSubcoreMesh(core_axis_name="c", subcore_axis_name="s")

    @functools.partial(
        pl.kernel,
        out_type=[jax.ShapeDtypeStruct((B, D1), _F32),
                  jax.ShapeDtypeStruct((B, D2), _F32)],
        mesh=mesh,
        scratch_types=[
            pltpu.VMEM((bpw,), jnp.int32),
            pltpu.VMEM((bpw,), jnp.int32),
            pltpu.VMEM((bpw, D1), _F32),
            pltpu.VMEM((bpw, D2), _F32),
            pltpu.SemaphoreType.DMA,
        ],
    )
    def gather(uidx_hbm, iidx_hbm, hu_hbm, ti_hbm, outu_hbm, outi_hbm,
               uidx_v, iidx_v, hurows_v, tirows_v, sem):
        wid = lax.axis_index("s") * nc + lax.axis_index("c")
        base = wid * bpw
        pltpu.sync_copy(uidx_hbm.at[pl.ds(base, bpw)], uidx_v)
        pltpu.sync_copy(iidx_hbm.at[pl.ds(base, bpw)], iidx_v)
        pltpu.async_copy(hu_hbm.at[uidx_v], hurows_v, sem).wait()
        pltpu.async_copy(ti_hbm.at[iidx_v], tirows_v, sem).wait()
        pltpu.sync_copy(hurows_v, outu_hbm.at[pl.ds(base, bpw)])
        pltpu.sync_copy(tirows_v, outi_hbm.at[pl.ds(base, bpw)])

    return gather


def _k5_body(hug_ref, tg_ref, vecs_ref, out_ref):
    d = 128
    t = tg_ref[...]
    hs = jax.nn.sigmoid(hug_ref[...] + t[:, :d])
    acc = _nt(hs, vecs_ref[_W_PRED:_W_PRED + 1, :])
    bp = vecs_ref[_B_PRED:_B_PRED + 1, 0:1]
    out_ref[...] = jax.nn.sigmoid(acc - t[:, d:d + 1] + bp)


def kernel(user, item, q, indicator, user_table, item_table, skill_table,
           W_stu, a_stu, W_item_stu, W_item_skill, a_item_stu, a_item_skill,
           a_map_stu, a_map_skill, W_skill_item, a_skill_item,
           W_fuse_stu, b_fuse_stu, W_fuse_item, b_fuse_item, W_pred, b_pred):
    U, d = user_table.shape
    I = item_table.shape[0]
    B = user.shape[0]
    BU = 2000

    vecs = jnp.stack([
        a_stu[d:], a_item_stu[d:], a_item_skill[d:], a_skill_item[d:],
        a_map_stu[:d], a_map_stu[d:], a_map_skill[:d], a_map_skill[d:],
        b_fuse_stu, b_fuse_item, W_pred[0],
        jnp.broadcast_to(b_pred, (d,)),
    ]).astype(_F32)

    nsteps = U // BU
    full = lambda shape: pl.BlockSpec(shape, lambda i: (0, 0))
    blk = lambda shape: pl.BlockSpec(shape, lambda i: (i, 0))
    h = BU // 2
    hu, t_item = pl.pallas_call(
        _k123_body,
        grid=(nsteps,),
        in_specs=[pl.BlockSpec((h, I), lambda i: (2 * i, 0)),
                  pl.BlockSpec((h, I), lambda i: (2 * i + 1, 0)),
                  blk((BU, d)), full((I, d)), full((d, d)),
                  full((I, d)), full((d, d)), full((d, d)), full((d, d)),
                  full((d, d)), full((d, 2 * d)), full((d, 2 * d)),
                  full((12, d))],
        out_specs=[blk((BU, d)), full((I, 2 * d))],
        out_shape=[jax.ShapeDtypeStruct((U, d), _F32),
                   jax.ShapeDtypeStruct((I, 2 * d), _F32)],
        scratch_shapes=[pltpu.VMEM((I, 2 * d), jnp.bfloat16),
                        pltpu.VMEM((2 * d, I), _F32)],
    )(indicator, indicator, user_table, item_table, skill_table, q,
      W_stu, W_item_stu, W_item_skill, W_skill_item, W_fuse_stu, W_fuse_item,
      vecs)

    hug, tg = _make_sc_gather(B, d, 2 * d)(
        user.astype(jnp.int32), item.astype(jnp.int32), hu, t_item)
    pred = pl.pallas_call(
        _k5_body,
        out_shape=jax.ShapeDtypeStruct((B, 1), _F32),
    )(hug, tg, vecs)
    return pred.reshape(B)
